# Initial kernel scaffold; baseline (speedup 1.0000x reference)
import jax
import jax.numpy as jnp
from jax.experimental import pallas as pl


def kernel(x, router_w, w1, w2, w3):
    raise NotImplementedError("write your pallas kernel here")


# speedup vs baseline: 1.4966x; 1.4966x over previous
import functools
import math

import jax
import jax.numpy as jnp
from jax import lax
from jax.experimental import pallas as pl
from jax.experimental.pallas import tpu as pltpu
from jax.experimental.pallas import tpu_sc as plsc

_TOPK = 2

_NC = 2
_NS = 16
_NW = _NC * _NS
_WR = 128


def _router_body(e, x_ref, rw_ref, tril_ref,
                 e0_ref, e1_ref, r0_ref, r1_ref, w0_ref, w1_ref, cnt_ref,
                 carry_ref):
    pid = pl.program_id(0)

    @pl.when(pid == 0)
    def _():
        carry_ref[...] = jnp.zeros_like(carry_ref)

    xb = x_ref[...]
    logits = jnp.dot(xb.astype(jnp.bfloat16), rw_ref[...].astype(jnp.bfloat16),
                     preferred_element_type=jnp.float32)
    m = jnp.max(logits, axis=-1, keepdims=True)
    p = jnp.exp(logits - m)
    p = p / jnp.sum(p, axis=-1, keepdims=True)

    eidx = lax.broadcasted_iota(jnp.int32, p.shape, 1)
    v0 = jnp.max(p, axis=-1, keepdims=True)
    i0 = jnp.min(jnp.where(p == v0, eidx, e), axis=-1, keepdims=True)
    p2 = jnp.where(eidx == i0, -1.0, p)
    v1 = jnp.max(p2, axis=-1, keepdims=True)
    i1 = jnp.min(jnp.where(p2 == v1, eidx, e), axis=-1, keepdims=True)
    s = v0 + v1
    w0 = v0 / s
    w1 = v1 / s

    oh0 = (eidx == i0).astype(jnp.float32)
    oh1 = (eidx == i1).astype(jnp.float32)
    oh = oh0 + oh1
    strict = jnp.dot(tril_ref[...], oh.astype(jnp.bfloat16),
                     preferred_element_type=jnp.float32)
    posf = carry_ref[0:1, 0:e] + strict
    r0 = jnp.sum(oh0 * posf, axis=-1, keepdims=True)
    r1 = jnp.sum(oh1 * posf, axis=-1, keepdims=True)

    e0_ref[0] = i0
    e1_ref[0] = i1
    r0_ref[0] = r0
    r1_ref[0] = r1
    w0_ref[0] = jnp.broadcast_to(w0, w0_ref.shape[1:])
    w1_ref[0] = jnp.broadcast_to(w1, w1_ref.shape[1:])
    carry_ref[0:1, 0:e] = carry_ref[0:1, 0:e] + jnp.sum(oh, axis=0,
                                                        keepdims=True)
    cnt_ref[...] = carry_ref[...]


def _run_router(x, router_w):
    a, d = x.shape
    e = router_w.shape[1]
    bt = min(1024, a)
    nb = a // bt
    tril = jnp.tril(jnp.ones((bt, bt), jnp.bfloat16), k=-1)
    out_shapes = (
        jax.ShapeDtypeStruct((nb, bt, 1), jnp.int32),
        jax.ShapeDtypeStruct((nb, bt, 1), jnp.int32),
        jax.ShapeDtypeStruct((nb, bt, 1), jnp.float32),
        jax.ShapeDtypeStruct((nb, bt, 1), jnp.float32),
        jax.ShapeDtypeStruct((nb, bt, _WR), jnp.float32),
        jax.ShapeDtypeStruct((nb, bt, _WR), jnp.float32),
        jax.ShapeDtypeStruct((8, 128), jnp.float32),
    )
    tok_spec = pl.BlockSpec((1, bt, 1), lambda i: (i, 0, 0))
    rep_spec = pl.BlockSpec((1, bt, _WR), lambda i: (i, 0, 0))
    return pl.pallas_call(
        functools.partial(_router_body, e),
        grid=(nb,),
        in_specs=[
            pl.BlockSpec((bt, d), lambda i: (i, 0)),
            pl.BlockSpec((d, e), lambda i: (0, 0)),
            pl.BlockSpec((bt, bt), lambda i: (0, 0)),
        ],
        out_specs=[tok_spec, tok_spec, tok_spec, tok_spec, rep_spec,
                   rep_spec, pl.BlockSpec((8, 128), lambda i: (0, 0))],
        out_shape=out_shapes,
        scratch_shapes=[pltpu.VMEM((8, 128), jnp.float32)],
        compiler_params=pltpu.CompilerParams(
            dimension_semantics=("arbitrary",)),
    )(x, router_w, tril)


def _pos_body(e, cap, cap2, e0_ref, e1_ref, r0_ref, r1_ref, cnt_ref,
              dst0_ref, dst1_ref):
    counts = cnt_ref[0:1, 0:e]
    adj = counts - (counts > 0).astype(jnp.float32)
    capm1 = jnp.float32(cap - 1)
    capf = jnp.float32(cap)

    def one(e_ref, r_ref, dst_ref):
        ei = e_ref[0]
        r = r_ref[0]
        eidx = lax.broadcasted_iota(jnp.int32, (ei.shape[0], e), 1)
        lt = (eidx < ei).astype(jnp.float32)
        off = jnp.sum(adj * lt, axis=-1, keepdims=True)
        cnt_e = jnp.sum((eidx == ei).astype(jnp.float32) * counts,
                        axis=-1, keepdims=True)
        posraw = r + off
        keep = (posraw < capm1) | (r == cnt_e - 1.0)
        pos = jnp.where(keep, jnp.minimum(posraw, capm1), capf)
        dst = ei.astype(jnp.float32) * jnp.float32(cap2) + pos
        dst_ref[0] = dst.astype(jnp.int32)

    one(e0_ref, r0_ref, dst0_ref)
    one(e1_ref, r1_ref, dst1_ref)


def _run_pos(e0, e1, r0, r1, counts, e, cap, cap2):
    nb, bt, _ = e0.shape
    tok_spec = pl.BlockSpec((1, bt, 1), lambda i: (i, 0, 0))
    return pl.pallas_call(
        functools.partial(_pos_body, e, cap, cap2),
        grid=(nb,),
        in_specs=[tok_spec, tok_spec, tok_spec, tok_spec,
                  pl.BlockSpec((8, 128), lambda i: (0, 0))],
        out_specs=[tok_spec, tok_spec],
        out_shape=(
            jax.ShapeDtypeStruct((nb, bt, 1), jnp.int32),
            jax.ShapeDtypeStruct((nb, bt, 1), jnp.int32),
        ),
    )(e0, e1, r0, r1, counts)


def _make_dispatch(a, d, n_slots, chunk):
    tok_per_w = a // _NW
    nch = tok_per_w // chunk
    mesh = plsc.VectorSubcoreMesh(core_axis_name="c", subcore_axis_name="s")

    @functools.partial(
        pl.kernel, mesh=mesh,
        out_type=[
            jax.ShapeDtypeStruct((n_slots, d), jnp.float32),
            jax.ShapeDtypeStruct((n_slots, _WR), jnp.float32),
        ],
        scratch_types=[
            pltpu.VMEM((chunk, d), jnp.float32),
            pltpu.VMEM((chunk,), jnp.int32),
            pltpu.VMEM((chunk,), jnp.int32),
            pltpu.VMEM((chunk, _WR), jnp.float32),
            pltpu.VMEM((chunk, _WR), jnp.float32),
            pltpu.SemaphoreType.DMA,
        ],
    )
    def dispatch(x_hbm, d0_hbm, d1_hbm, w0_hbm, w1_hbm,
                 xbuf_hbm, wbuf_hbm,
                 rows_v, idx0_v, idx1_v, w0_v, w1_v, sem):
        wid = lax.axis_index("s") * _NC + lax.axis_index("c")
        for ch in range(nch):
            base = wid * tok_per_w + ch * chunk
            pltpu.sync_copy(x_hbm.at[pl.ds(base, chunk)], rows_v)
            pltpu.sync_copy(d0_hbm.at[wid, ch], idx0_v)
            pltpu.sync_copy(d1_hbm.at[wid, ch], idx1_v)
            pltpu.sync_copy(w0_hbm.at[wid, ch], w0_v)
            pltpu.sync_copy(w1_hbm.at[wid, ch], w1_v)
            cps = [
                pltpu.async_copy(rows_v, xbuf_hbm.at[idx0_v], sem),
                pltpu.async_copy(rows_v, xbuf_hbm.at[idx1_v], sem),
                pltpu.async_copy(w0_v, wbuf_hbm.at[idx0_v], sem),
                pltpu.async_copy(w1_v, wbuf_hbm.at[idx1_v], sem),
            ]
            for cp in cps:
                cp.wait()

    return dispatch


def _mlp_body(nh, cap, xb_ref, w1_ref, w3_ref, w2_ref, wcol_ref, out_ref,
              xbf_ref):
    hstep = pl.program_id(1)

    @pl.when(hstep == 0)
    def _():
        xbf_ref[...] = xb_ref[0].astype(jnp.bfloat16)

    xb = xbf_ref[...]
    pre1 = jnp.dot(xb, w1_ref[0].astype(jnp.bfloat16),
                   preferred_element_type=jnp.float32)
    pre3 = jnp.dot(xb, w3_ref[0].astype(jnp.bfloat16),
                   preferred_element_type=jnp.float32)
    g = pre1 * jax.nn.sigmoid(pre1) * pre3
    contrib = jnp.dot(g.astype(jnp.bfloat16), w2_ref[0].astype(jnp.bfloat16),
                      preferred_element_type=jnp.float32)

    @pl.when(hstep == 0)
    def _():
        out_ref[0] = jnp.zeros_like(out_ref[0])

    out_ref[0] += contrib

    @pl.when(hstep == nh - 1)
    def _():
        cap2 = out_ref.shape[1]
        row = lax.broadcasted_iota(jnp.int32, (cap2, 1), 0)
        scale = jnp.where(row < cap, wcol_ref[0][:, 0:1], 0.0)
        out_ref[0] = out_ref[0] * scale


def _run_mlp(xbuf, w1, w3, w2, wbuf, cap):
    e, cap2, d = xbuf.shape
    h = w1.shape[2]
    bh = 256 if h % 256 == 0 else h
    nh = h // bh
    return pl.pallas_call(
        functools.partial(_mlp_body, nh, cap),
        grid=(e, nh),
        in_specs=[
            pl.BlockSpec((1, cap2, d), lambda i, j: (i, 0, 0)),
            pl.BlockSpec((1, d, bh), lambda i, j: (i, 0, j)),
            pl.BlockSpec((1, d, bh), lambda i, j: (i, 0, j)),
            pl.BlockSpec((1, bh, d), lambda i, j: (i, j, 0)),
            pl.BlockSpec((1, cap2, _WR), lambda i, j: (i, 0, 0)),
        ],
        out_specs=pl.BlockSpec((1, cap2, d), lambda i, j: (i, 0, 0)),
        out_shape=jax.ShapeDtypeStruct((e, cap2, d), jnp.float32),
        scratch_shapes=[pltpu.VMEM((cap2, d), jnp.bfloat16)],
        compiler_params=pltpu.CompilerParams(
            dimension_semantics=("arbitrary", "arbitrary")),
    )(xbuf, w1, w3, w2, wbuf)


def _make_combine(a, d, n_slots, chunk):
    tok_per_w = a // _NW
    nch = tok_per_w // chunk
    nvec = d // _NS
    mesh = plsc.VectorSubcoreMesh(core_axis_name="c", subcore_axis_name="s")

    @functools.partial(
        pl.kernel, mesh=mesh,
        out_type=jax.ShapeDtypeStruct((a, d), jnp.float32),
        scratch_types=[
            pltpu.VMEM((chunk, d), jnp.float32),
            pltpu.VMEM((chunk, d), jnp.float32),
            pltpu.VMEM((chunk, d), jnp.float32),
            pltpu.VMEM((chunk,), jnp.int32),
            pltpu.VMEM((chunk,), jnp.int32),
            pltpu.SemaphoreType.DMA,
        ],
    )
    def combine(slots_hbm, d0_hbm, d1_hbm, y_hbm,
                rows0_v, rows1_v, ybuf_v, idx0_v, idx1_v, sem):
        wid = lax.axis_index("s") * _NC + lax.axis_index("c")
        for ch in range(nch):
            base = wid * tok_per_w + ch * chunk
            pltpu.sync_copy(d0_hbm.at[wid, ch], idx0_v)
            pltpu.sync_copy(d1_hbm.at[wid, ch], idx1_v)
            cp0 = pltpu.async_copy(slots_hbm.at[idx0_v], rows0_v, sem)
            cp1 = pltpu.async_copy(slots_hbm.at[idx1_v], rows1_v, sem)
            cp0.wait()
            cp1.wait()

            def tok_body(i, _):
                def vec_body(j, _):
                    sl = pl.ds(j * _NS, _NS)
                    ybuf_v[i, sl] = rows0_v[i, sl] + rows1_v[i, sl]
                    return 0
                return lax.fori_loop(0, nvec, vec_body, 0)

            lax.fori_loop(0, chunk, tok_body, 0)
            pltpu.sync_copy(ybuf_v, y_hbm.at[pl.ds(base, chunk)])

    return combine


def kernel(x, router_w, w1, w2, w3):
    a, d = x.shape
    e = router_w.shape[1]
    cap = max(1, int(math.ceil(1.25 * a * _TOPK / e)))
    cap2 = cap + 8
    n_slots = e * cap2

    e0, e1, r0, r1, w0r, w1r, counts = _run_router(x, router_w)
    dst0, dst1 = _run_pos(e0, e1, r0, r1, counts, e, cap, cap2)
    dst0 = dst0.reshape(a)
    dst1 = dst1.reshape(a)

    chunk = 64
    d0_disp = dst0.reshape(_NW, -1, chunk)
    d1_disp = dst1.reshape(_NW, -1, chunk)
    w0_disp = w0r.reshape(a, _WR).reshape(_NW, -1, chunk, _WR)
    w1_disp = w1r.reshape(a, _WR).reshape(_NW, -1, chunk, _WR)

    xbuf, wbuf = _make_dispatch(a, d, n_slots, chunk)(
        x, d0_disp, d1_disp, w0_disp, w1_disp)

    yslots = _run_mlp(xbuf.reshape(e, cap2, d), w1, w3, w2,
                      wbuf.reshape(e, cap2, _WR), cap)

    chunk2 = 32
    d0_comb = dst0.reshape(_NW, -1, chunk2)
    d1_comb = dst1.reshape(_NW, -1, chunk2)
    y = _make_combine(a, d, n_slots, chunk2)(
        yslots.reshape(n_slots, d), d0_comb, d1_comb)
    return y


# speedup vs baseline: 2.1771x; 1.4547x over previous
import functools
import math

import jax
import jax.numpy as jnp
from jax import lax
from jax.experimental import pallas as pl
from jax.experimental.pallas import tpu as pltpu
from jax.experimental.pallas import tpu_sc as plsc

_TOPK = 2

_NC = 2
_NS = 16
_NW = _NC * _NS
_RB = 128


def _router_body(e, x_ref, rw_ref, tril_ref,
                 e0_ref, e1_ref, r0_ref, r1_ref, w0_ref, w1_ref, cnt_ref,
                 carry_ref):
    pid = pl.program_id(0)

    @pl.when(pid == 0)
    def _():
        carry_ref[...] = jnp.zeros_like(carry_ref)

    xb = x_ref[...]
    logits = jnp.dot(xb.astype(jnp.bfloat16), rw_ref[...].astype(jnp.bfloat16),
                     preferred_element_type=jnp.float32)
    m = jnp.max(logits, axis=-1, keepdims=True)
    p = jnp.exp(logits - m)
    p = p / jnp.sum(p, axis=-1, keepdims=True)

    eidx = lax.broadcasted_iota(jnp.int32, p.shape, 1)
    v0 = jnp.max(p, axis=-1, keepdims=True)
    i0 = jnp.min(jnp.where(p == v0, eidx, e), axis=-1, keepdims=True)
    p2 = jnp.where(eidx == i0, -1.0, p)
    v1 = jnp.max(p2, axis=-1, keepdims=True)
    i1 = jnp.min(jnp.where(p2 == v1, eidx, e), axis=-1, keepdims=True)
    s = v0 + v1
    w0 = v0 / s
    w1 = v1 / s

    oh0 = (eidx == i0).astype(jnp.float32)
    oh1 = (eidx == i1).astype(jnp.float32)
    oh = oh0 + oh1
    strict = jnp.dot(tril_ref[...], oh.astype(jnp.bfloat16),
                     preferred_element_type=jnp.float32)
    posf = carry_ref[0:1, 0:e] + strict
    r0 = jnp.sum(oh0 * posf, axis=-1, keepdims=True)
    r1 = jnp.sum(oh1 * posf, axis=-1, keepdims=True)

    e0_ref[0] = i0
    e1_ref[0] = i1
    r0_ref[0] = r0
    r1_ref[0] = r1
    w0_ref[0] = w0
    w1_ref[0] = w1
    carry_ref[0:1, 0:e] = carry_ref[0:1, 0:e] + jnp.sum(oh, axis=0,
                                                        keepdims=True)
    cnt_ref[...] = carry_ref[...]


def _run_router(x, router_w):
    a, d = x.shape
    e = router_w.shape[1]
    bt = min(1024, a)
    nb = a // bt
    tril = jnp.tril(jnp.ones((bt, bt), jnp.bfloat16), k=-1)
    out_shapes = (
        jax.ShapeDtypeStruct((nb, bt, 1), jnp.int32),
        jax.ShapeDtypeStruct((nb, bt, 1), jnp.int32),
        jax.ShapeDtypeStruct((nb, bt, 1), jnp.float32),
        jax.ShapeDtypeStruct((nb, bt, 1), jnp.float32),
        jax.ShapeDtypeStruct((nb, bt, 1), jnp.float32),
        jax.ShapeDtypeStruct((nb, bt, 1), jnp.float32),
        jax.ShapeDtypeStruct((8, 128), jnp.float32),
    )
    tok_spec = pl.BlockSpec((1, bt, 1), lambda i: (i, 0, 0))
    return pl.pallas_call(
        functools.partial(_router_body, e),
        grid=(nb,),
        in_specs=[
            pl.BlockSpec((bt, d), lambda i: (i, 0)),
            pl.BlockSpec((d, e), lambda i: (0, 0)),
            pl.BlockSpec((bt, bt), lambda i: (0, 0)),
        ],
        out_specs=[tok_spec, tok_spec, tok_spec, tok_spec, tok_spec,
                   tok_spec, pl.BlockSpec((8, 128), lambda i: (0, 0))],
        out_shape=out_shapes,
        scratch_shapes=[pltpu.VMEM((8, 128), jnp.float32)],
        compiler_params=pltpu.CompilerParams(
            dimension_semantics=("arbitrary",)),
    )(x, router_w, tril)


def _pos_body(e, cap, cap2, e0_ref, e1_ref, r0_ref, r1_ref, w0_ref, w1_ref,
              cnt_ref,
              ds0_ref, ds1_ref, dg0_ref, dg1_ref, wr0_ref, wr1_ref, blk_ref):
    counts = cnt_ref[0:1, 0:e]
    adj = counts - (counts > 0).astype(jnp.float32)
    capm1 = jnp.float32(cap - 1)
    capf = jnp.float32(cap)
    cap2f = jnp.float32(cap2)

    def one(e_ref, r_ref, w_ref, ds_ref, dg_ref, wr_ref):
        ei = e_ref[0]
        r = r_ref[0]
        w = w_ref[0]
        eidx = lax.broadcasted_iota(jnp.int32, (ei.shape[0], e), 1)
        lt = (eidx < ei).astype(jnp.float32)
        off = jnp.sum(adj * lt, axis=-1, keepdims=True)
        eeq = (eidx == ei).astype(jnp.float32)
        cnt_e = jnp.sum(eeq * counts, axis=-1, keepdims=True)
        posraw = r + off
        keep = (posraw < capm1) | (r == cnt_e - 1.0)
        pos = jnp.minimum(posraw, capm1)
        trash = capf + (r - jnp.floor(r * 0.125) * 8.0)
        win = jnp.minimum(off + cnt_e, capf) - 1.0
        ebase = ei.astype(jnp.float32) * cap2f
        ds_ref[0] = (ebase + jnp.where(keep, pos, trash)).astype(jnp.int32)
        dg_ref[0] = (ebase + jnp.where(keep, pos, win)).astype(jnp.int32)
        weff = jnp.where(keep, w, 0.0)
        wr_ref[0] = jnp.broadcast_to(weff, wr_ref.shape[1:])

    one(e0_ref, r0_ref, w0_ref, ds0_ref, dg0_ref, wr0_ref)
    one(e1_ref, r1_ref, w1_ref, ds1_ref, dg1_ref, wr1_ref)

    adjb = jnp.broadcast_to(adj, (e, e))
    cntb = jnp.broadcast_to(counts, (e, e))
    sub = lax.broadcasted_iota(jnp.int32, (e, e), 0)
    lane = lax.broadcasted_iota(jnp.int32, (e, e), 1)
    offc = jnp.sum(adjb * (lane < sub).astype(jnp.float32), axis=-1,
                   keepdims=True)
    cntc = jnp.sum(cntb * (lane == sub).astype(jnp.float32), axis=-1,
                   keepdims=True)
    startc = jnp.minimum(offc, capm1)
    endc = jnp.where(cntc > 0.0, jnp.minimum(offc + cntc, capf), 0.0)
    rbf = jnp.float32(_RB)
    lo = jnp.floor(startc / rbf)
    hi = jnp.floor((endc + rbf - 1.0) / rbf)
    lane128 = lax.broadcasted_iota(jnp.int32, (e, 128), 1)
    vals = jnp.where(lane128 == 0, lo, jnp.where(lane128 == 1, hi, 0.0))
    blk_ref[...] = vals.astype(jnp.int32)


def _run_pos(e0, e1, r0, r1, w0, w1, counts, e, cap, cap2):
    nb, bt, _ = e0.shape
    tok_spec = pl.BlockSpec((1, bt, 1), lambda i: (i, 0, 0))
    rep_spec = pl.BlockSpec((1, bt, _NS), lambda i: (i, 0, 0))
    return pl.pallas_call(
        functools.partial(_pos_body, e, cap, cap2),
        grid=(nb,),
        in_specs=[tok_spec, tok_spec, tok_spec, tok_spec, tok_spec,
                  tok_spec, pl.BlockSpec((8, 128), lambda i: (0, 0))],
        out_specs=[tok_spec, tok_spec, tok_spec, tok_spec, rep_spec,
                   rep_spec, pl.BlockSpec((e, 128), lambda i: (0, 0))],
        out_shape=(
            jax.ShapeDtypeStruct((nb, bt, 1), jnp.int32),
            jax.ShapeDtypeStruct((nb, bt, 1), jnp.int32),
            jax.ShapeDtypeStruct((nb, bt, 1), jnp.int32),
            jax.ShapeDtypeStruct((nb, bt, 1), jnp.int32),
            jax.ShapeDtypeStruct((nb, bt, _NS), jnp.float32),
            jax.ShapeDtypeStruct((nb, bt, _NS), jnp.float32),
            jax.ShapeDtypeStruct((e, 128), jnp.int32),
        ),
    )(e0, e1, r0, r1, w0, w1, counts)


def _make_dispatch(a, d, n_slots, chunk):
    tok_per_w = a // _NW
    nch = tok_per_w // chunk
    mesh = plsc.VectorSubcoreMesh(core_axis_name="c", subcore_axis_name="s")

    @functools.partial(
        pl.kernel, mesh=mesh,
        out_type=jax.ShapeDtypeStruct((n_slots, d), jnp.float32),
        scratch_types=[
            pltpu.VMEM((chunk, d), jnp.float32),
            pltpu.VMEM((chunk, d), jnp.float32),
            pltpu.VMEM((nch, chunk), jnp.int32),
            pltpu.VMEM((nch, chunk), jnp.int32),
            pltpu.SemaphoreType.DMA,
            pltpu.SemaphoreType.DMA,
        ],
    )
    def dispatch(x_hbm, d0_hbm, d1_hbm, xbuf_hbm,
                 rows_a, rows_b, idx0_v, idx1_v, sem_l, sem_s):
        wid = lax.axis_index("s") * _NC + lax.axis_index("c")
        base_w = wid * tok_per_w
        pltpu.sync_copy(d0_hbm.at[wid], idx0_v)
        pltpu.sync_copy(d1_hbm.at[wid], idx1_v)
        bufs = (rows_a, rows_b)
        pltpu.async_copy(x_hbm.at[pl.ds(base_w, chunk)], bufs[0],
                         sem_l).wait()
        for g in range(nch):
            b = g % 2
            nxt = None
            if g + 1 < nch:
                nxt = pltpu.async_copy(
                    x_hbm.at[pl.ds(base_w + (g + 1) * chunk, chunk)],
                    bufs[1 - b], sem_l)
            s0 = pltpu.async_copy(bufs[b], xbuf_hbm.at[idx0_v.at[g]], sem_s)
            s1 = pltpu.async_copy(bufs[b], xbuf_hbm.at[idx1_v.at[g]], sem_s)
            s0.wait()
            s1.wait()
            if nxt is not None:
                nxt.wait()

    return dispatch


def _mlp_body(e, nh, sinfo_ref, xb_ref, w1_ref, w3_ref, w2_ref, out_ref,
              xbf_ref):
    ei = pl.program_id(0)
    hstep = pl.program_id(1)
    lo = sinfo_ref[ei]
    hi = sinfo_ref[e + ei]
    w1b = w1_ref[0].astype(jnp.bfloat16)
    w3b = w3_ref[0].astype(jnp.bfloat16)
    w2b = w2_ref[0].astype(jnp.bfloat16)

    @pl.when(hstep == 0)
    def _():
        def cast_body(i, carry):
            rs = pl.ds(i * _RB, _RB)
            xbf_ref[rs, :] = xb_ref[0, rs, :].astype(jnp.bfloat16)
            return carry
        lax.fori_loop(lo, hi, cast_body, 0)

    def blk_body(i, carry):
        rs = pl.ds(i * _RB, _RB)
        xr = xbf_ref[rs, :]
        aa = jnp.dot(xr, w1b, preferred_element_type=jnp.float32)
        bb = jnp.dot(xr, w3b, preferred_element_type=jnp.float32)
        g = aa * jax.nn.sigmoid(aa) * bb
        cc = jnp.dot(g.astype(jnp.bfloat16), w2b,
                     preferred_element_type=jnp.float32)
        prev = out_ref[0, rs, :]
        out_ref[0, rs, :] = jnp.where(hstep == 0, cc, prev + cc)
        return carry

    lax.fori_loop(lo, hi, blk_body, 0)


def _run_mlp(xbuf, w1, w3, w2, sinfo):
    e, cap2, d = xbuf.shape
    h = w1.shape[2]
    bh = 256 if h % 256 == 0 else h
    nh = h // bh
    grid_spec = pltpu.PrefetchScalarGridSpec(
        num_scalar_prefetch=1,
        grid=(e, nh),
        in_specs=[
            pl.BlockSpec((1, cap2, d), lambda i, j, *_: (i, 0, 0)),
            pl.BlockSpec((1, d, bh), lambda i, j, *_: (i, 0, j)),
            pl.BlockSpec((1, d, bh), lambda i, j, *_: (i, 0, j)),
            pl.BlockSpec((1, bh, d), lambda i, j, *_: (i, j, 0)),
        ],
        out_specs=pl.BlockSpec((1, cap2, d), lambda i, j, *_: (i, 0, 0)),
        scratch_shapes=[pltpu.VMEM((cap2, d), jnp.bfloat16)],
    )
    return pl.pallas_call(
        functools.partial(_mlp_body, e, nh),
        grid_spec=grid_spec,
        out_shape=jax.ShapeDtypeStruct((e, cap2, d), jnp.float32),
        compiler_params=pltpu.CompilerParams(
            dimension_semantics=("arbitrary", "arbitrary")),
    )(sinfo, xbuf, w1, w3, w2)


def _make_combine(a, d, n_slots, chunk):
    tok_per_w = a // _NW
    nch = tok_per_w // chunk
    nvec = d // _NS
    mesh = plsc.VectorSubcoreMesh(core_axis_name="c", subcore_axis_name="s")

    @functools.partial(
        pl.kernel, mesh=mesh,
        out_type=jax.ShapeDtypeStruct((a, d), jnp.float32),
        scratch_types=[
            pltpu.VMEM((chunk, d), jnp.float32),
            pltpu.VMEM((chunk, d), jnp.float32),
            pltpu.VMEM((chunk, d), jnp.float32),
            pltpu.VMEM((chunk, d), jnp.float32),
            pltpu.VMEM((chunk, d), jnp.float32),
            pltpu.VMEM((chunk, d), jnp.float32),
            pltpu.VMEM((nch, chunk), jnp.int32),
            pltpu.VMEM((nch, chunk), jnp.int32),
            pltpu.VMEM((nch, chunk, _NS), jnp.float32),
            pltpu.VMEM((nch, chunk, _NS), jnp.float32),
            pltpu.SemaphoreType.DMA,
            pltpu.SemaphoreType.DMA,
            pltpu.SemaphoreType.DMA,
            pltpu.SemaphoreType.DMA,
        ],
    )
    def combine(slots_hbm, d0_hbm, d1_hbm, w0_hbm, w1_hbm, y_hbm,
                r0a, r0b, r1a, r1b, ya, yb, idx0_v, idx1_v, w0_v, w1_v,
                sem_ga, sem_gb, sem_ya, sem_yb):
        wid = lax.axis_index("s") * _NC + lax.axis_index("c")
        base_w = wid * tok_per_w
        pltpu.sync_copy(d0_hbm.at[wid], idx0_v)
        pltpu.sync_copy(d1_hbm.at[wid], idx1_v)
        pltpu.sync_copy(w0_hbm.at[wid], w0_v)
        pltpu.sync_copy(w1_hbm.at[wid], w1_v)
        r0s = (r0a, r0b)
        r1s = (r1a, r1b)
        ys = (ya, yb)
        sg = (sem_ga, sem_gb)
        sy = (sem_ya, sem_yb)
        pltpu.async_copy(slots_hbm.at[idx0_v.at[0]], r0s[0], sg[0])
        pltpu.async_copy(slots_hbm.at[idx1_v.at[0]], r1s[0], sg[0])

        def pair_body(g2, carry):
            for b in (0, 1):
                ch = g2 * 2 + b

                @pl.when(ch + 1 < nch)
                def _():
                    pltpu.async_copy(slots_hbm.at[idx0_v.at[ch + 1]],
                                     r0s[1 - b], sg[1 - b])
                    pltpu.async_copy(slots_hbm.at[idx1_v.at[ch + 1]],
                                     r1s[1 - b], sg[1 - b])

                pltpu.make_async_copy(slots_hbm.at[idx0_v.at[ch]], r0s[b],
                                      sg[b]).wait()
                pltpu.make_async_copy(slots_hbm.at[idx1_v.at[ch]], r1s[b],
                                      sg[b]).wait()

                @pl.when(ch >= 2)
                def _():
                    pltpu.make_async_copy(
                        y_hbm.at[pl.ds(base_w + (ch - 2) * chunk, chunk)],
                        ys[b], sy[b]).wait()

                def tok_body(i, carry2):
                    wv0 = w0_v[ch, i, :]
                    wv1 = w1_v[ch, i, :]
                    for j in range(nvec):
                        sl = pl.ds(j * _NS, _NS)
                        ys[b][i, sl] = (r0s[b][i, sl] * wv0
                                        + r1s[b][i, sl] * wv1)
                    return carry2

                lax.fori_loop(0, chunk, tok_body, 0)
                pltpu.async_copy(ys[b],
                                 y_hbm.at[pl.ds(base_w + ch * chunk, chunk)],
                                 sy[b])
            return carry

        lax.fori_loop(0, nch // 2, pair_body, 0)
        for g in (nch - 2, nch - 1):
            b = g % 2
            pltpu.make_async_copy(
                y_hbm.at[pl.ds(base_w + g * chunk, chunk)], ys[b],
                sy[b]).wait()

    return combine


def kernel(x, router_w, w1, w2, w3):
    a, d = x.shape
    e = router_w.shape[1]
    cap = max(1, int(math.ceil(1.25 * a * _TOPK / e)))
    cap2 = cap + 8
    n_slots = e * cap2

    e0, e1, r0, r1, w0, w1r_, counts = _run_router(x, router_w)
    ds0, ds1, dg0, dg1, wr0, wr1, blk = _run_pos(
        e0, e1, r0, r1, w0, w1r_, counts, e, cap, cap2)
    ds0 = ds0.reshape(a)
    ds1 = ds1.reshape(a)
    dg0 = dg0.reshape(a)
    dg1 = dg1.reshape(a)
    sinfo = jnp.concatenate([blk[:, 0], blk[:, 1]])

    chunk = 16
    xbuf = _make_dispatch(a, d, n_slots, chunk)(
        x, ds0.reshape(_NW, -1, chunk), ds1.reshape(_NW, -1, chunk))

    yslots = _run_mlp(xbuf.reshape(e, cap2, d), w1, w3, w2, sinfo)

    chunk2 = 8
    y = _make_combine(a, d, n_slots, chunk2)(
        yslots.reshape(n_slots, d),
        dg0.reshape(_NW, -1, chunk2), dg1.reshape(_NW, -1, chunk2),
        wr0.reshape(a, _NS).reshape(_NW, -1, chunk2, _NS),
        wr1.reshape(a, _NS).reshape(_NW, -1, chunk2, _NS))
    return y
